# baseline (speedup 1.0000x reference)
import math

import jax
import jax.numpy as jnp
import numpy as np
from jax import lax
from jax.experimental import pallas as pl
from jax.experimental.pallas import tpu as pltpu
from jax.experimental.pallas import tpu_sc as plsc

D_MODEL = 768
_SCALE = math.sqrt(float(D_MODEL))
_LANES = 16
_CHUNK = 32


def _pos_encoding(seq_len: int, d_model: int) -> np.ndarray:
    pos = np.arange(seq_len, dtype=np.float32)[:, None]
    i = np.arange(d_model, dtype=np.float32)[None, :]
    angle_rates = 1.0 / np.power(10000.0, (2.0 * np.floor(i / 2.0)) / d_model)
    angles = pos * angle_rates
    pe = np.zeros((seq_len, d_model), dtype=np.float32)
    pe[:, 0::2] = np.sin(angles[:, 0::2])
    pe[:, 1::2] = np.cos(angles[:, 1::2])
    return pe


def _make_sc_call(B: int, T: int, V: int, D: int):
    info = plsc.get_sparse_core_info()
    NC, NS = info.num_cores, info.num_subcores
    NW = NC * NS
    assert T % NW == 0
    t_per_w = T // NW
    assert t_per_w % _CHUNK == 0
    halves = t_per_w // _CHUNK
    n_chunks = B * halves

    mesh = plsc.VectorSubcoreMesh(core_axis_name="c", subcore_axis_name="s")

    @jax.jit
    def call(token_ids, table, pe):
        @pl.kernel(
            mesh=mesh,
            out_type=jax.ShapeDtypeStruct((B * T, D), jnp.float32),
            scratch_types=[
                pltpu.VMEM((B, t_per_w), jnp.int32),
                pltpu.VMEM((t_per_w, D), jnp.float32),
                pltpu.VMEM((_CHUNK, D), jnp.float32),
                pltpu.VMEM((_CHUNK, D), jnp.float32),
                pltpu.SemaphoreType.DMA,
                pltpu.SemaphoreType.DMA,
                pltpu.SemaphoreType.DMA,
                pltpu.SemaphoreType.DMA,
            ],
        )
        def k(idx_hbm, table_hbm, pe_hbm, out_hbm,
              idx_v, pe_v, g0, g1, sg0, sg1, ss0, ss1):
            wid = lax.axis_index("s") * NC + lax.axis_index("c")
            t0 = wid * t_per_w
            for b in range(B):
                pltpu.sync_copy(idx_hbm.at[b, pl.ds(t0, t_per_w)], idx_v.at[b])

            gbuf = (g0, g1)
            gsem = (sg0, sg1)
            ssem = (ss0, ss1)

            def gather_start(c):
                b, half = divmod(c, halves)
                k_ = c % 2
                idx = idx_v.at[b, pl.ds(half * _CHUNK, _CHUNK)]
                return pltpu.async_copy(table_hbm.at[idx], gbuf[k_], gsem[k_])

            def store_start(c):
                b, half = divmod(c, halves)
                k_ = c % 2
                dst = out_hbm.at[pl.ds(b * T + t0 + half * _CHUNK, _CHUNK)]
                return pltpu.async_copy(gbuf[k_], dst, ssem[k_])

            h_g = gather_start(0)
            pltpu.sync_copy(pe_hbm.at[pl.ds(t0, t_per_w)], pe_v)

            h_s = [None] * n_chunks
            for c in range(n_chunks):
                h_g.wait()
                if c + 1 < n_chunks:
                    if c >= 1:
                        h_s[c - 1].wait()
                    h_g = gather_start(c + 1)
                g = gbuf[c % 2]
                off = (c % halves) * _CHUNK

                def body(r, _):
                    for j in range(D // _LANES):
                        sl = pl.ds(j * _LANES, _LANES)
                        g[r, sl] = g[r, sl] * _SCALE + pe_v[off + r, sl]
                    return _

                lax.fori_loop(0, _CHUNK, body, None)
                h_s[c] = store_start(c)
            h_s[n_chunks - 2].wait()
            h_s[n_chunks - 1].wait()

        return k(token_ids, table, pe)

    return call


def kernel(token_ids, embedding_table):
    B, T = token_ids.shape
    V, D = embedding_table.shape
    pe = jnp.asarray(_pos_encoding(T, D))
    call = _make_sc_call(B, T, V, D)
    out = call(token_ids, embedding_table, pe)
    return out.reshape(B, T, D)
